# baseline (speedup 1.0000x reference)
import jax
import jax.numpy as jnp
from jax.experimental import pallas as pl

_SLOPE = (1.0 / 8.0 + 1.0 / 3.0) / 2.0


def _body(x_ref, w1_ref, w2_ref, b_ref, o_ref):
    x = x_ref[...]
    h1 = jnp.where(x >= 0, x, _SLOPE * x)
    h2 = jnp.where(h1 >= 0, h1, _SLOPE * h1)
    acc = jnp.dot(h1, w1_ref[...], preferred_element_type=jnp.float32)
    acc += jnp.dot(h2, w2_ref[...], preferred_element_type=jnp.float32)
    o_ref[...] = acc + b_ref[...]


def kernel(node_feats, edge_feats, edge_index, W1, b1, lW1, lb1, W2, b2, lW2, lb2, oW, ob):
    n, d = node_feats.shape
    h = oW.shape[1]
    block = 1000
    grid = (n // block,)
    w1 = oW[:d]
    w2 = oW[d:]
    b = ob.reshape(1, h)
    return pl.pallas_call(
        _body,
        grid=grid,
        in_specs=[
            pl.BlockSpec((block, d), lambda i: (i, 0)),
            pl.BlockSpec((d, h), lambda i: (0, 0)),
            pl.BlockSpec((d, h), lambda i: (0, 0)),
            pl.BlockSpec((1, h), lambda i: (0, 0)),
        ],
        out_specs=pl.BlockSpec((block, h), lambda i: (i, 0)),
        out_shape=jax.ShapeDtypeStruct((n, h), jnp.float32),
    )(node_feats, w1, w2, b)
